# Initial kernel scaffold; baseline (speedup 1.0000x reference)
import jax
import jax.numpy as jnp
from jax.experimental import pallas as pl


def kernel(z, edge_index, W1, b1, W2, b2):
    raise NotImplementedError("write your pallas kernel here")


# speedup vs baseline: 8.4143x; 8.4143x over previous
import functools

import jax
import jax.numpy as jnp
from jax import lax
from jax.experimental import pallas as pl
from jax.experimental.pallas import tpu as pltpu
from jax.experimental.pallas import tpu_sc as plsc

_CHUNK = 1024
_SUB = 128
_H = 16


def _tc_project(z_ref, wc_ref, b1_ref, a_ref, b_ref):
    r = jnp.dot(z_ref[...], wc_ref[...], preferred_element_type=jnp.float32)
    a_ref[...] = r[:, :_H] + b1_ref[...]
    b_ref[...] = r[:, _H:]


def _sc_score(nw, cpw, a_hbm, b_hbm, src_hbm, dst_hbm, w2_hbm, b2_hbm, out_hbm,
              idx_s, idx_d, rows_a, rows_b, out_c, w2_v, b2_v, sem):
    wid = lax.axis_index("s") * 2 + lax.axis_index("c")
    pltpu.sync_copy(w2_hbm, w2_v)
    pltpu.sync_copy(b2_hbm, b2_v)
    w2r = w2_v[...]
    b2r = b2_v[...]
    iota16 = lax.iota(jnp.int32, 16)
    w2bc = [w2r[jnp.full((16,), k, jnp.int32)] for k in range(_H)]
    cols = [jnp.full((16,), k, jnp.int32) for k in range(_H)]
    nsub = _CHUNK // _SUB

    def chunk_body(ci, carry):
        c = wid * cpw + ci
        pltpu.sync_copy(src_hbm.at[pl.ds(c * nsub, nsub)], idx_s)
        pltpu.sync_copy(dst_hbm.at[pl.ds(c * nsub, nsub)], idx_d)
        cps = []
        for j in range(nsub):
            cps.append(pltpu.async_copy(
                a_hbm.at[idx_s.at[j]], rows_a.at[pl.ds(j * _SUB, _SUB)], sem))
            cps.append(pltpu.async_copy(
                b_hbm.at[idx_d.at[j]], rows_b.at[pl.ds(j * _SUB, _SUB)], sem))
        for cp in cps:
            cp.wait()

        def group_body(g, carry2):
            erow = g * 16 + iota16
            acc = b2r
            for k in range(_H):
                va = plsc.load_gather(rows_a, [erow, cols[k]])
                vb = plsc.load_gather(rows_b, [erow, cols[k]])
                acc = acc + jnp.maximum(va + vb, 0.0) * w2bc[k]
            out_c[pl.ds(g * 16, 16)] = acc
            return carry2

        lax.fori_loop(0, _CHUNK // 16, group_body, 0)
        pltpu.sync_copy(out_c, out_hbm.at[pl.ds(c * _CHUNK, _CHUNK)])
        return carry

    lax.fori_loop(0, cpw, chunk_body, 0)


def kernel(z, edge_index, W1, b1, W2, b2):
    n_nodes, d = z.shape
    e = edge_index.shape[1]

    wc = jnp.concatenate([W1[:, :d].T, W1[:, d:].T], axis=1)
    a_t, b_t = pl.pallas_call(
        _tc_project,
        out_shape=[
            jax.ShapeDtypeStruct((n_nodes, _H), jnp.float32),
            jax.ShapeDtypeStruct((n_nodes, _H), jnp.float32),
        ],
    )(z, wc, b1.reshape(1, _H))

    info = plsc.get_sparse_core_info()
    nc, ns = info.num_cores, info.num_subcores
    nw = nc * ns
    cpw = -(-e // (nw * _CHUNK))
    e_pad = nw * cpw * _CHUNK

    si = edge_index[0].astype(jnp.int32)
    di = edge_index[1].astype(jnp.int32)
    pad = e_pad - e
    if pad:
        zeros = jnp.zeros((pad,), jnp.int32)
        si = jnp.concatenate([si, zeros])
        di = jnp.concatenate([di, zeros])
    si = si.reshape(e_pad // _SUB, _SUB)
    di = di.reshape(e_pad // _SUB, _SUB)

    w2v = W2.reshape(_H)
    b2v = jnp.full((_H,), b2[0], jnp.float32)

    mesh = plsc.VectorSubcoreMesh(core_axis_name="c", subcore_axis_name="s")
    score = pl.kernel(
        functools.partial(_sc_score, nw, cpw),
        out_type=jax.ShapeDtypeStruct((e_pad,), jnp.float32),
        mesh=mesh,
        compiler_params=pltpu.CompilerParams(
            needs_layout_passes=False, use_tc_tiling_on_sc=False),
        scratch_types=[
            pltpu.VMEM((_CHUNK // _SUB, _SUB), jnp.int32),
            pltpu.VMEM((_CHUNK // _SUB, _SUB), jnp.int32),
            pltpu.VMEM((_CHUNK, _H), jnp.float32),
            pltpu.VMEM((_CHUNK, _H), jnp.float32),
            pltpu.VMEM((_CHUNK,), jnp.float32),
            pltpu.VMEM((_H,), jnp.float32),
            pltpu.VMEM((_H,), jnp.float32),
            pltpu.SemaphoreType.DMA,
        ],
    )
    out_pad = score(a_t, b_t, si, di, w2v, b2v)
    return out_pad[:e]


# speedup vs baseline: 11.4472x; 1.3604x over previous
import functools

import jax
import jax.numpy as jnp
from jax import lax
from jax.experimental import pallas as pl
from jax.experimental.pallas import tpu as pltpu
from jax.experimental.pallas import tpu_sc as plsc

_CHUNK = 1024
_SUB = 128
_H = 16


def _tc_project(z_ref, wc_ref, b1_ref, a_ref, b_ref):
    r = jnp.dot(z_ref[...], wc_ref[...], preferred_element_type=jnp.float32)
    a_ref[...] = r[:, :_H] + b1_ref[...]
    b_ref[...] = r[:, _H:]


def _sc_score(nw, cpw, n_nodes, a_hbm, b_hbm, src_hbm, dst_hbm, w2_hbm, b2_hbm,
              out_hbm, idx_s, idx_d, rows_a, rows_b, out_c, w2_v, b2_v, a_sh,
              b_sh, sem):
    sid = lax.axis_index("s")
    wid = sid * 2 + lax.axis_index("c")
    seg = n_nodes // 16
    pltpu.sync_copy(a_hbm.at[pl.ds(sid * seg, seg)], a_sh.at[pl.ds(sid * seg, seg)])
    pltpu.sync_copy(b_hbm.at[pl.ds(sid * seg, seg)], b_sh.at[pl.ds(sid * seg, seg)])
    pltpu.sync_copy(w2_hbm, w2_v)
    pltpu.sync_copy(b2_hbm, b2_v)
    plsc.subcore_barrier()
    w2r = w2_v[...]
    b2r = b2_v[...]
    iota16 = lax.iota(jnp.int32, 16)
    w2bc = [w2r[jnp.full((16,), k, jnp.int32)] for k in range(_H)]
    cols = [jnp.full((16,), k, jnp.int32) for k in range(_H)]
    nsub = _CHUNK // _SUB

    def chunk_body(ci, carry):
        c = wid * cpw + ci
        pltpu.sync_copy(src_hbm.at[pl.ds(c * nsub, nsub)], idx_s)
        pltpu.sync_copy(dst_hbm.at[pl.ds(c * nsub, nsub)], idx_d)
        cps = []
        for j in range(nsub):
            cps.append(pltpu.async_copy(
                a_sh.at[idx_s.at[j]], rows_a.at[pl.ds(j * _SUB, _SUB)], sem))
            cps.append(pltpu.async_copy(
                b_sh.at[idx_d.at[j]], rows_b.at[pl.ds(j * _SUB, _SUB)], sem))
        for cp in cps:
            cp.wait()

        def group_body(g, carry2):
            erow = g * 16 + iota16
            acc = b2r
            for k in range(_H):
                va = plsc.load_gather(rows_a, [erow, cols[k]])
                vb = plsc.load_gather(rows_b, [erow, cols[k]])
                acc = acc + jnp.maximum(va + vb, 0.0) * w2bc[k]
            out_c[pl.ds(g * 16, 16)] = acc
            return carry2

        lax.fori_loop(0, _CHUNK // 16, group_body, 0)
        pltpu.sync_copy(out_c, out_hbm.at[pl.ds(c * _CHUNK, _CHUNK)])
        return carry

    lax.fori_loop(0, cpw, chunk_body, 0)


def kernel(z, edge_index, W1, b1, W2, b2):
    n_nodes, d = z.shape
    e = edge_index.shape[1]

    wc = jnp.concatenate([W1[:, :d].T, W1[:, d:].T], axis=1)
    a_t, b_t = pl.pallas_call(
        _tc_project,
        out_shape=[
            jax.ShapeDtypeStruct((n_nodes, _H), jnp.float32),
            jax.ShapeDtypeStruct((n_nodes, _H), jnp.float32),
        ],
    )(z, wc, b1.reshape(1, _H))

    info = plsc.get_sparse_core_info()
    nc, ns = info.num_cores, info.num_subcores
    nw = nc * ns
    cpw = -(-e // (nw * _CHUNK))
    e_pad = nw * cpw * _CHUNK

    si = edge_index[0].astype(jnp.int32)
    di = edge_index[1].astype(jnp.int32)
    pad = e_pad - e
    if pad:
        zeros = jnp.zeros((pad,), jnp.int32)
        si = jnp.concatenate([si, zeros])
        di = jnp.concatenate([di, zeros])
    si = si.reshape(e_pad // _SUB, _SUB)
    di = di.reshape(e_pad // _SUB, _SUB)

    w2v = W2.reshape(_H)
    b2v = jnp.full((_H,), b2[0], jnp.float32)

    mesh = plsc.VectorSubcoreMesh(core_axis_name="c", subcore_axis_name="s")
    score = pl.kernel(
        functools.partial(_sc_score, nw, cpw, n_nodes),
        out_type=jax.ShapeDtypeStruct((e_pad,), jnp.float32),
        mesh=mesh,
        compiler_params=pltpu.CompilerParams(
            needs_layout_passes=False, use_tc_tiling_on_sc=False),
        scratch_types=[
            pltpu.VMEM((_CHUNK // _SUB, _SUB), jnp.int32),
            pltpu.VMEM((_CHUNK // _SUB, _SUB), jnp.int32),
            pltpu.VMEM((_CHUNK, _H), jnp.float32),
            pltpu.VMEM((_CHUNK, _H), jnp.float32),
            pltpu.VMEM((_CHUNK,), jnp.float32),
            pltpu.VMEM((_H,), jnp.float32),
            pltpu.VMEM((_H,), jnp.float32),
            pltpu.VMEM_SHARED((n_nodes, _H), jnp.float32),
            pltpu.VMEM_SHARED((n_nodes, _H), jnp.float32),
            pltpu.SemaphoreType.DMA,
        ],
    )
    out_pad = score(a_t, b_t, si, di, w2v, b2v)
    return out_pad[:e]


# speedup vs baseline: 12.4440x; 1.0871x over previous
import functools

import jax
import jax.numpy as jnp
from jax import lax
from jax.experimental import pallas as pl
from jax.experimental.pallas import tpu as pltpu
from jax.experimental.pallas import tpu_sc as plsc

_CHUNK = 1024
_SUB = 128
_H = 16


def _tc_project(z_ref, wc_ref, b1_ref, a_ref, b_ref):
    r = jnp.dot(z_ref[...], wc_ref[...], preferred_element_type=jnp.float32)
    a_ref[...] = r[:, :_H] + b1_ref[...]
    b_ref[...] = r[:, _H:]


def _sc_score(nw, cpw, n_nodes, a_hbm, b_hbm, src_hbm, dst_hbm, w2_hbm, b2_hbm,
              out_hbm, idx_s, idx_d, rows_a, rows_b, out_c, w2_v, b2_v, a_sh,
              b_sh, sem):
    sid = lax.axis_index("s")
    wid = sid * 2 + lax.axis_index("c")
    seg = n_nodes // 16
    pltpu.sync_copy(a_hbm.at[pl.ds(sid * seg, seg)], a_sh.at[pl.ds(sid * seg, seg)])
    pltpu.sync_copy(b_hbm.at[pl.ds(sid * seg, seg)], b_sh.at[pl.ds(sid * seg, seg)])
    pltpu.sync_copy(w2_hbm, w2_v)
    pltpu.sync_copy(b2_hbm, b2_v)
    plsc.subcore_barrier()
    w2r = w2_v[...]
    b2r = b2_v[...]
    iota16 = lax.iota(jnp.int32, 16)
    w2bc = [w2r[jnp.full((16,), k, jnp.int32)] for k in range(_H)]
    cols = [jnp.full((16,), k, jnp.int32) for k in range(_H)]
    nsub = _CHUNK // _SUB

    def copy_idx(ci, parity):
        c = wid * cpw + ci
        ioff = parity * nsub
        pltpu.sync_copy(src_hbm.at[pl.ds(c * nsub, nsub)],
                        idx_s.at[pl.ds(ioff, nsub)])
        pltpu.sync_copy(dst_hbm.at[pl.ds(c * nsub, nsub)],
                        idx_d.at[pl.ds(ioff, nsub)])

    def fire_gathers(parity):
        roff = parity * _CHUNK
        ioff = parity * nsub
        for j in range(nsub):
            pltpu.async_copy(a_sh.at[idx_s.at[ioff + j]],
                             rows_a.at[pl.ds(roff + j * _SUB, _SUB)], sem)
            pltpu.async_copy(b_sh.at[idx_d.at[ioff + j]],
                             rows_b.at[pl.ds(roff + j * _SUB, _SUB)], sem)

    def wait_gathers(parity):
        roff = parity * _CHUNK
        for j in range(nsub):
            pltpu.make_async_copy(a_hbm.at[pl.ds(0, _SUB)],
                                  rows_a.at[pl.ds(roff + j * _SUB, _SUB)],
                                  sem).wait()
            pltpu.make_async_copy(a_hbm.at[pl.ds(0, _SUB)],
                                  rows_b.at[pl.ds(roff + j * _SUB, _SUB)],
                                  sem).wait()

    def compute(ci, parity):
        c = wid * cpw + ci
        roff = parity * _CHUNK

        def group_body(g, carry2):
            erow = roff + g * 16 + iota16
            acc = b2r
            for k in range(_H):
                va = plsc.load_gather(rows_a, [erow, cols[k]])
                vb = plsc.load_gather(rows_b, [erow, cols[k]])
                acc = acc + jnp.maximum(va + vb, 0.0) * w2bc[k]
            out_c[pl.ds(g * 16, 16)] = acc
            return carry2

        lax.fori_loop(0, _CHUNK // 16, group_body, 0)
        pltpu.sync_copy(out_c, out_hbm.at[pl.ds(c * _CHUNK, _CHUNK)])

    copy_idx(0, 0)
    fire_gathers(0)

    def chunk_body(ci, carry):
        p = lax.rem(ci, 2)
        copy_idx(ci + 1, 1 - p)
        fire_gathers(1 - p)
        wait_gathers(p)
        compute(ci, p)
        return carry

    lax.fori_loop(0, cpw - 1, chunk_body, 0)
    p_last = lax.rem(jnp.int32(cpw - 1), 2)
    wait_gathers(p_last)
    compute(cpw - 1, p_last)


def kernel(z, edge_index, W1, b1, W2, b2):
    n_nodes, d = z.shape
    e = edge_index.shape[1]

    wc = jnp.concatenate([W1[:, :d].T, W1[:, d:].T], axis=1)
    a_t, b_t = pl.pallas_call(
        _tc_project,
        out_shape=[
            jax.ShapeDtypeStruct((n_nodes, _H), jnp.float32),
            jax.ShapeDtypeStruct((n_nodes, _H), jnp.float32),
        ],
    )(z, wc, b1.reshape(1, _H))

    info = plsc.get_sparse_core_info()
    nc, ns = info.num_cores, info.num_subcores
    nw = nc * ns
    cpw = -(-e // (nw * _CHUNK))
    e_pad = nw * cpw * _CHUNK

    si = edge_index[0].astype(jnp.int32)
    di = edge_index[1].astype(jnp.int32)
    pad = e_pad - e
    if pad:
        zeros = jnp.zeros((pad,), jnp.int32)
        si = jnp.concatenate([si, zeros])
        di = jnp.concatenate([di, zeros])
    si = si.reshape(e_pad // _SUB, _SUB)
    di = di.reshape(e_pad // _SUB, _SUB)

    w2v = W2.reshape(_H)
    b2v = jnp.full((_H,), b2[0], jnp.float32)

    mesh = plsc.VectorSubcoreMesh(core_axis_name="c", subcore_axis_name="s")
    score = pl.kernel(
        functools.partial(_sc_score, nw, cpw, n_nodes),
        out_type=jax.ShapeDtypeStruct((e_pad,), jnp.float32),
        mesh=mesh,
        compiler_params=pltpu.CompilerParams(
            needs_layout_passes=False, use_tc_tiling_on_sc=False),
        scratch_types=[
            pltpu.VMEM((2 * _CHUNK // _SUB, _SUB), jnp.int32),
            pltpu.VMEM((2 * _CHUNK // _SUB, _SUB), jnp.int32),
            pltpu.VMEM((2 * _CHUNK, _H), jnp.float32),
            pltpu.VMEM((2 * _CHUNK, _H), jnp.float32),
            pltpu.VMEM((_CHUNK,), jnp.float32),
            pltpu.VMEM((_H,), jnp.float32),
            pltpu.VMEM((_H,), jnp.float32),
            pltpu.VMEM_SHARED((n_nodes, _H), jnp.float32),
            pltpu.VMEM_SHARED((n_nodes, _H), jnp.float32),
            pltpu.SemaphoreType.DMA,
        ],
    )
    out_pad = score(a_t, b_t, si, di, w2v, b2v)
    return out_pad[:e]


# speedup vs baseline: 13.6608x; 1.0978x over previous
import functools

import jax
import jax.numpy as jnp
from jax import lax
from jax.experimental import pallas as pl
from jax.experimental.pallas import tpu as pltpu
from jax.experimental.pallas import tpu_sc as plsc

_CHUNK = 1024
_SUB = 128
_H = 16


def _tc_project(z_ref, wc_ref, b1_ref, a_ref, b_ref):
    r = jnp.dot(z_ref[...], wc_ref[...], preferred_element_type=jnp.float32)
    a_ref[...] = r[:, :_H] + b1_ref[...]
    b_ref[...] = r[:, _H:]


def _sc_score(nw, cpw, n_nodes, a_hbm, b_hbm, src_hbm, dst_hbm, w2_hbm, b2_hbm,
              out_hbm, idx_s, idx_d, rows_a, rows_b, out_c, w2_m, b2_v, a_sh,
              b_sh, sem):
    sid = lax.axis_index("s")
    wid = sid * 2 + lax.axis_index("c")
    seg = n_nodes // 16
    pltpu.sync_copy(a_hbm.at[pl.ds(sid * seg, seg)], a_sh.at[pl.ds(sid * seg, seg)])
    pltpu.sync_copy(b_hbm.at[pl.ds(sid * seg, seg)], b_sh.at[pl.ds(sid * seg, seg)])
    pltpu.sync_copy(w2_hbm, w2_m)
    pltpu.sync_copy(b2_hbm, b2_v)
    plsc.subcore_barrier()
    b2r = b2_v[...]
    iota16 = lax.iota(jnp.int32, 16)
    nsub = _CHUNK // _SUB

    def copy_idx(ci, parity):
        c = wid * cpw + ci
        ioff = parity * nsub
        pltpu.sync_copy(src_hbm.at[pl.ds(c * nsub, nsub)],
                        idx_s.at[pl.ds(ioff, nsub)])
        pltpu.sync_copy(dst_hbm.at[pl.ds(c * nsub, nsub)],
                        idx_d.at[pl.ds(ioff, nsub)])

    def fire_gathers(parity):
        roff = parity * _CHUNK
        ioff = parity * nsub
        for j in range(nsub):
            pltpu.async_copy(a_sh.at[idx_s.at[ioff + j]],
                             rows_a.at[pl.ds(roff + j * _SUB, _SUB)], sem)
            pltpu.async_copy(b_sh.at[idx_d.at[ioff + j]],
                             rows_b.at[pl.ds(roff + j * _SUB, _SUB)], sem)

    def wait_gathers(parity):
        roff = parity * _CHUNK
        for j in range(nsub):
            pltpu.make_async_copy(a_hbm.at[pl.ds(0, _SUB)],
                                  rows_a.at[pl.ds(roff + j * _SUB, _SUB)],
                                  sem).wait()
            pltpu.make_async_copy(a_hbm.at[pl.ds(0, _SUB)],
                                  rows_b.at[pl.ds(roff + j * _SUB, _SUB)],
                                  sem).wait()

    def compute(ci, parity):
        c = wid * cpw + ci
        roff = parity * _CHUNK

        def group_body(g, carry2):
            erow = roff + g * 16 + iota16
            accs = [b2r, jnp.zeros((16,), jnp.float32),
                    jnp.zeros((16,), jnp.float32), jnp.zeros((16,), jnp.float32)]
            for k in range(_H):
                ck = jnp.full((16,), k, jnp.int32)
                va = plsc.load_gather(rows_a, [erow, ck])
                vb = plsc.load_gather(rows_b, [erow, ck])
                accs[k % 4] = accs[k % 4] + jnp.maximum(va + vb, 0.0) * w2_m[k]
            out_c[pl.ds(g * 16, 16)] = (accs[0] + accs[1]) + (accs[2] + accs[3])
            return carry2

        lax.fori_loop(0, _CHUNK // 16, group_body, 0)
        pltpu.sync_copy(out_c, out_hbm.at[pl.ds(c * _CHUNK, _CHUNK)])

    copy_idx(0, 0)
    fire_gathers(0)

    def chunk_body(ci, carry):
        p = lax.rem(ci, 2)
        copy_idx(ci + 1, 1 - p)
        fire_gathers(1 - p)
        wait_gathers(p)
        compute(ci, p)
        return carry

    lax.fori_loop(0, cpw - 1, chunk_body, 0)
    p_last = lax.rem(jnp.int32(cpw - 1), 2)
    wait_gathers(p_last)
    compute(cpw - 1, p_last)


def kernel(z, edge_index, W1, b1, W2, b2):
    n_nodes, d = z.shape
    e = edge_index.shape[1]

    wc = jnp.concatenate([W1[:, :d].T, W1[:, d:].T], axis=1)
    a_t, b_t = pl.pallas_call(
        _tc_project,
        out_shape=[
            jax.ShapeDtypeStruct((n_nodes, _H), jnp.float32),
            jax.ShapeDtypeStruct((n_nodes, _H), jnp.float32),
        ],
    )(z, wc, b1.reshape(1, _H))

    info = plsc.get_sparse_core_info()
    nc, ns = info.num_cores, info.num_subcores
    nw = nc * ns
    cpw = -(-e // (nw * _CHUNK))
    e_pad = nw * cpw * _CHUNK

    si = edge_index[0].astype(jnp.int32)
    di = edge_index[1].astype(jnp.int32)
    pad = e_pad - e
    if pad:
        zeros = jnp.zeros((pad,), jnp.int32)
        si = jnp.concatenate([si, zeros])
        di = jnp.concatenate([di, zeros])
    si = si.reshape(e_pad // _SUB, _SUB)
    di = di.reshape(e_pad // _SUB, _SUB)

    w2m = jnp.broadcast_to(W2.reshape(_H, 1), (_H, 16)).astype(jnp.float32)
    b2v = jnp.full((_H,), b2[0], jnp.float32)

    mesh = plsc.VectorSubcoreMesh(core_axis_name="c", subcore_axis_name="s")
    score = pl.kernel(
        functools.partial(_sc_score, nw, cpw, n_nodes),
        out_type=jax.ShapeDtypeStruct((e_pad,), jnp.float32),
        mesh=mesh,
        compiler_params=pltpu.CompilerParams(
            needs_layout_passes=False, use_tc_tiling_on_sc=False),
        scratch_types=[
            pltpu.VMEM((2 * _CHUNK // _SUB, _SUB), jnp.int32),
            pltpu.VMEM((2 * _CHUNK // _SUB, _SUB), jnp.int32),
            pltpu.VMEM((2 * _CHUNK, _H), jnp.float32),
            pltpu.VMEM((2 * _CHUNK, _H), jnp.float32),
            pltpu.VMEM((_CHUNK,), jnp.float32),
            pltpu.VMEM((_H, 16), jnp.float32),
            pltpu.VMEM((_H,), jnp.float32),
            pltpu.VMEM_SHARED((n_nodes, _H), jnp.float32),
            pltpu.VMEM_SHARED((n_nodes, _H), jnp.float32),
            pltpu.SemaphoreType.DMA,
        ],
    )
    out_pad = score(a_t, b_t, si, di, w2m, b2v)
    return out_pad[:e]
